# baseline (speedup 1.0000x reference)
import functools

import jax
import jax.numpy as jnp
from jax import lax
from jax.experimental import pallas as pl
from jax.experimental.pallas import tpu as pltpu
from jax.experimental.pallas import tpu_sc as plsc

D = 2048
T = 512
NC, NS = 2, 16
NW = NC * NS
S = 4


def _make_row_gather(n_rows, d, ch):
    per_w = n_rows // NW
    nchunks = per_w // ch
    assert n_rows % NW == 0 and per_w % ch == 0 and nchunks % 2 == 0

    mesh = plsc.VectorSubcoreMesh(core_axis_name="c", subcore_axis_name="s")

    @functools.partial(
        pl.kernel,
        out_type=jax.ShapeDtypeStruct((n_rows, d), jnp.float32),
        mesh=mesh,
        scratch_types=[
            pltpu.VMEM((nchunks, ch), jnp.int32),
            pltpu.VMEM((ch, d), jnp.float32),
            pltpu.VMEM((ch, d), jnp.float32),
            pltpu.SemaphoreType.DMA,
        ],
    )
    def gather(src_hbm, idx_hbm, out_hbm, idx_v, buf0, buf1, gsem):
        wid = lax.axis_index("s") * NC + lax.axis_index("c")
        base = wid * per_w
        pltpu.sync_copy(idx_hbm.at[wid], idx_v)
        bufs = (buf0, buf1)
        pltpu.async_copy(src_hbm.at[idx_v.at[0]], buf0, gsem)

        def pair(g, carry):
            for b in range(2):
                j = g * 2 + b
                cur, nxt = bufs[b], bufs[1 - b]
                pltpu.make_async_copy(src_hbm.at[idx_v.at[0]], cur, gsem).wait()

                @pl.when(j + 1 < nchunks)
                def _():
                    pltpu.async_copy(src_hbm.at[idx_v.at[j + 1]], nxt, gsem)

                pltpu.sync_copy(cur, out_hbm.at[pl.ds(base + j * ch, ch)])
            return carry

        lax.fori_loop(0, nchunks // 2, pair, 0)

    return gather


def _make_row_gather_into(n_seg_rows, d, ch, row_base):
    per_w = n_seg_rows // NW
    nchunks = per_w // ch
    assert n_seg_rows % NW == 0 and per_w % ch == 0 and nchunks % 2 == 0

    mesh = plsc.VectorSubcoreMesh(core_axis_name="c", subcore_axis_name="s")

    @functools.partial(
        pl.kernel,
        out_type=(),
        mesh=mesh,
        scratch_types=[
            pltpu.VMEM((nchunks, ch), jnp.int32),
            pltpu.VMEM((ch, d), jnp.float32),
            pltpu.VMEM((ch, d), jnp.float32),
            pltpu.SemaphoreType.DMA,
        ],
    )
    def gather(src_hbm, idx_hbm, dst_hbm, idx_v, buf0, buf1, gsem):
        wid = lax.axis_index("s") * NC + lax.axis_index("c")
        base = row_base + wid * per_w
        pltpu.sync_copy(idx_hbm.at[wid], idx_v)
        bufs = (buf0, buf1)
        pltpu.async_copy(src_hbm.at[idx_v.at[0]], buf0, gsem)

        def pair(g, carry):
            for b in range(2):
                j = g * 2 + b
                cur, nxt = bufs[b], bufs[1 - b]
                pltpu.make_async_copy(src_hbm.at[idx_v.at[0]], cur, gsem).wait()

                @pl.when(j + 1 < nchunks)
                def _():
                    pltpu.async_copy(src_hbm.at[idx_v.at[j + 1]], nxt, gsem)

                pltpu.sync_copy(cur, dst_hbm.at[pl.ds(base + j * ch, ch)])
            return carry

        lax.fori_loop(0, nchunks // 2, pair, 0)

    return gather


def _routed_matmul(flags, x, w0, w1):
    n = x.shape[0]
    ntiles = n // T

    def body(flags_ref, x_ref, w0_ref, w1_ref, o_ref):
        f = flags_ref[pl.program_id(0)]
        xb = x_ref[...].astype(jnp.bfloat16)

        @pl.when(f == 0)
        def _():
            o_ref[...] = lax.dot_general(
                xb, w0_ref[...], (((1,), (1,)), ((), ())),
                preferred_element_type=jnp.float32)

        @pl.when(f != 0)
        def _():
            o_ref[...] = lax.dot_general(
                xb, w1_ref[...], (((1,), (1,)), ((), ())),
                preferred_element_type=jnp.float32)

    grid_spec = pltpu.PrefetchScalarGridSpec(
        num_scalar_prefetch=1,
        grid=(ntiles,),
        in_specs=[
            pl.BlockSpec((T, D), lambda t, flags: (t, 0)),
            pl.BlockSpec((D, D), lambda t, flags: (0, 0)),
            pl.BlockSpec((D, D), lambda t, flags: (0, 0)),
        ],
        out_specs=pl.BlockSpec((T, D), lambda t, flags: (t, 0)),
    )
    return pl.pallas_call(
        body,
        grid_spec=grid_spec,
        out_shape=jax.ShapeDtypeStruct((n, D), jnp.float32),
    )(flags, x, w0, w1)


def kernel(input_ids, role_mask, table, W0, W1):
    b, l = input_ids.shape
    n = b * l
    seg = n // S
    npad = seg + T
    ntiles = npad // T

    ids = input_ids.reshape(S, seg).astype(jnp.int32)
    is0 = role_mask.reshape(S, seg) == 0
    i0 = is0.astype(jnp.int32)
    r0 = jnp.cumsum(i0, axis=1) - 1
    r1 = jnp.cumsum(1 - i0, axis=1) - 1
    c0 = jnp.sum(i0, axis=1, keepdims=True)
    start1 = (c0 // T) * T + T
    pos = jnp.where(is0, r0, start1 + r1).astype(jnp.int32)

    pos_flat = (pos + npad * jnp.arange(S, dtype=jnp.int32)[:, None]).reshape(-1)
    slot_ids = jnp.zeros((S * npad,), jnp.int32).at[pos_flat].set(ids.reshape(-1))
    slot_ids = slot_ids.reshape(S, npad)
    flags = (jnp.arange(ntiles, dtype=jnp.int32)[None, :] >= (c0 // T + 1)).astype(jnp.int32)

    gather_in = _make_row_gather(npad, D, 24)
    w0b = W0.astype(jnp.bfloat16)
    w1b = W1.astype(jnp.bfloat16)

    out_ref = jax.new_ref(lax.empty((n, D), jnp.float32))
    for s in range(S):
        x_s = gather_in(table, slot_ids[s].reshape(NW, -1, 24))
        y_s = _routed_matmul(flags[s], x_s, w0b, w1b)
        gather_out = _make_row_gather_into(seg, D, 16, s * seg)
        gather_out(y_s, pos[s].reshape(NW, -1, 16), out_ref)

    return out_ref[...].reshape(b, l, D)


# speedup vs baseline: 1.1884x; 1.1884x over previous
import functools

import jax
import jax.numpy as jnp
from jax import lax
from jax.experimental import pallas as pl
from jax.experimental.pallas import tpu as pltpu
from jax.experimental.pallas import tpu_sc as plsc

D = 2048
T = 512
NC, NS = 2, 16
NW = NC * NS
S = 2


def _make_row_gather(n_rows, d, ch):
    per_w = n_rows // NW
    nchunks = per_w // ch
    assert n_rows % NW == 0 and per_w % ch == 0 and ch % 8 == 0

    mesh = plsc.VectorSubcoreMesh(core_axis_name="c", subcore_axis_name="s")

    @functools.partial(
        pl.kernel,
        out_type=jax.ShapeDtypeStruct((n_rows, d), jnp.float32),
        mesh=mesh,
        scratch_types=[
            pltpu.VMEM((nchunks, ch), jnp.int32),
            pltpu.VMEM((ch, d), jnp.float32),
            pltpu.VMEM((ch, d), jnp.float32),
            pltpu.SemaphoreType.DMA,
        ],
    )
    def gather(src_hbm, idx_hbm, out_hbm, idx_v, buf0, buf1, gsem):
        wid = lax.axis_index("s") * NC + lax.axis_index("c")
        base = wid * per_w
        pltpu.sync_copy(idx_hbm.at[wid], idx_v)
        bufs = (buf0, buf1)
        pltpu.async_copy(src_hbm.at[idx_v.at[0]], buf0, gsem)

        def pair(g, carry):
            for b in range(2):
                j = g * 2 + b
                cur, nxt = bufs[b], bufs[1 - b]
                pltpu.make_async_copy(src_hbm.at[idx_v.at[0]], cur, gsem).wait()

                @pl.when(j + 1 < nchunks)
                def _():
                    pltpu.async_copy(src_hbm.at[idx_v.at[j + 1]], nxt, gsem)

                pltpu.sync_copy(cur, out_hbm.at[pl.ds(base + j * ch, ch)])
            return carry

        lax.fori_loop(0, nchunks // 2, pair, 0)
        if nchunks % 2:
            j = nchunks - 1
            cur = bufs[j % 2]
            pltpu.make_async_copy(src_hbm.at[idx_v.at[0]], cur, gsem).wait()
            pltpu.sync_copy(cur, out_hbm.at[pl.ds(base + j * ch, ch)])

    return gather


def _make_row_gather_into(n_seg_rows, d, ch, row_base):
    per_w = n_seg_rows // NW
    nchunks = per_w // ch
    assert n_seg_rows % NW == 0 and per_w % ch == 0 and ch % 8 == 0

    mesh = plsc.VectorSubcoreMesh(core_axis_name="c", subcore_axis_name="s")

    @functools.partial(
        pl.kernel,
        out_type=(),
        mesh=mesh,
        scratch_types=[
            pltpu.VMEM((nchunks, ch), jnp.int32),
            pltpu.VMEM((ch, d), jnp.float32),
            pltpu.VMEM((ch, d), jnp.float32),
            pltpu.SemaphoreType.DMA,
        ],
    )
    def gather(src_hbm, idx_hbm, dst_hbm, idx_v, buf0, buf1, gsem):
        wid = lax.axis_index("s") * NC + lax.axis_index("c")
        base = row_base + wid * per_w
        pltpu.sync_copy(idx_hbm.at[wid], idx_v)
        bufs = (buf0, buf1)
        pltpu.async_copy(src_hbm.at[idx_v.at[0]], buf0, gsem)

        def pair(g, carry):
            for b in range(2):
                j = g * 2 + b
                cur, nxt = bufs[b], bufs[1 - b]
                pltpu.make_async_copy(src_hbm.at[idx_v.at[0]], cur, gsem).wait()

                @pl.when(j + 1 < nchunks)
                def _():
                    pltpu.async_copy(src_hbm.at[idx_v.at[j + 1]], nxt, gsem)

                pltpu.sync_copy(cur, dst_hbm.at[pl.ds(base + j * ch, ch)])
            return carry

        lax.fori_loop(0, nchunks // 2, pair, 0)
        if nchunks % 2:
            j = nchunks - 1
            cur = bufs[j % 2]
            pltpu.make_async_copy(src_hbm.at[idx_v.at[0]], cur, gsem).wait()
            pltpu.sync_copy(cur, dst_hbm.at[pl.ds(base + j * ch, ch)])

    return gather


def _routed_matmul(flags, x, w0, w1):
    n = x.shape[0]
    ntiles = n // T

    def body(flags_ref, x_ref, w0_ref, w1_ref, o_ref):
        f = flags_ref[pl.program_id(0)]
        xb = x_ref[...].astype(jnp.bfloat16)

        @pl.when(f == 0)
        def _():
            o_ref[...] = lax.dot_general(
                xb, w0_ref[...], (((1,), (1,)), ((), ())),
                preferred_element_type=jnp.float32)

        @pl.when(f != 0)
        def _():
            o_ref[...] = lax.dot_general(
                xb, w1_ref[...], (((1,), (1,)), ((), ())),
                preferred_element_type=jnp.float32)

    grid_spec = pltpu.PrefetchScalarGridSpec(
        num_scalar_prefetch=1,
        grid=(ntiles,),
        in_specs=[
            pl.BlockSpec((T, D), lambda t, flags: (t, 0)),
            pl.BlockSpec((D, D), lambda t, flags: (0, 0)),
            pl.BlockSpec((D, D), lambda t, flags: (0, 0)),
        ],
        out_specs=pl.BlockSpec((T, D), lambda t, flags: (t, 0)),
    )
    return pl.pallas_call(
        body,
        grid_spec=grid_spec,
        out_shape=jax.ShapeDtypeStruct((n, D), jnp.float32),
    )(flags, x, w0, w1)


def kernel(input_ids, role_mask, table, W0, W1):
    b, l = input_ids.shape
    n = b * l
    seg = n // S
    npad = seg + T
    ntiles = npad // T

    ids = input_ids.reshape(S, seg).astype(jnp.int32)
    is0 = role_mask.reshape(S, seg) == 0
    i0 = is0.astype(jnp.int32)
    r0 = jnp.cumsum(i0, axis=1) - 1
    r1 = jnp.cumsum(1 - i0, axis=1) - 1
    c0 = jnp.sum(i0, axis=1, keepdims=True)
    start1 = (c0 // T) * T + T
    pos = jnp.where(is0, r0, start1 + r1).astype(jnp.int32)

    pos_flat = (pos + npad * jnp.arange(S, dtype=jnp.int32)[:, None]).reshape(-1)
    slot_ids = jnp.zeros((S * npad,), jnp.int32).at[pos_flat].set(ids.reshape(-1))
    slot_ids = slot_ids.reshape(S, npad)
    flags = (jnp.arange(ntiles, dtype=jnp.int32)[None, :] >= (c0 // T + 1)).astype(jnp.int32)

    gather_in = _make_row_gather(npad, D, 16)
    w0b = W0.astype(jnp.bfloat16)
    w1b = W1.astype(jnp.bfloat16)

    out_ref = jax.new_ref(lax.empty((n, D), jnp.float32))
    for s in range(S):
        x_s = gather_in(table, slot_ids[s].reshape(NW, -1, 16))
        y_s = _routed_matmul(flags[s], x_s, w0b, w1b)
        gather_out = _make_row_gather_into(seg, D, 16, s * seg)
        gather_out(y_s, pos[s].reshape(NW, -1, 16), out_ref)

    return out_ref[...].reshape(b, l, D)
